# Initial kernel scaffold; baseline (speedup 1.0000x reference)
import jax
import jax.numpy as jnp
from jax.experimental import pallas as pl


def kernel(logits):
    raise NotImplementedError("write your pallas kernel here")


# speedup vs baseline: 42.6735x; 42.6735x over previous
import jax
import jax.numpy as jnp
from jax.experimental import pallas as pl
from jax.experimental.pallas import tpu as pltpu
from jax.experimental.pallas import tpu_sc as plsc

B = 64
V = 1_000_000
K = 50
R = 4
L = 64
CHUNK = 32_768
NCH = 256
STEPS = 31
NC = STEPS * NCH
NBLK = 2 * NC
INT_MAX = 2**31 - 1
NEG_INF = float("-inf")


def _k1_body(x_ref, a_ref, b_ref, t_ref):
    i = pl.program_id(0)

    def emit(x):
        x3 = x.reshape(B, NCH, 2 * L)
        t_ref[...] = x3
        a_ref[...] = jnp.max(x3[:, :, :L], axis=2)
        b_ref[...] = jnp.max(x3[:, :, L:], axis=2)

    @pl.when(i < STEPS - 1)
    def _():
        emit(x_ref[...])

    @pl.when(i == STEPS - 1)
    def _():
        gcol = CHUNK * i + jax.lax.broadcasted_iota(jnp.int32, (B, CHUNK), 1)
        emit(jnp.where(gcol < V, x_ref[...], NEG_INF))


def _k1_block_max(logits):
    return pl.pallas_call(
        _k1_body,
        grid=(STEPS,),
        in_specs=[pl.BlockSpec((B, CHUNK), lambda i: (0, i))],
        out_specs=[
            pl.BlockSpec((B, NCH), lambda i: (0, i)),
            pl.BlockSpec((B, NCH), lambda i: (0, i)),
            pl.BlockSpec((B, NCH, 2 * L), lambda i: (0, i, 0)),
        ],
        out_shape=[
            jax.ShapeDtypeStruct((B, NC), jnp.float32),
            jax.ShapeDtypeStruct((B, NC), jnp.float32),
            jax.ShapeDtypeStruct((B, NC, 2 * L), jnp.float32),
        ],
        interpret=False,
    )(logits)


def _k2_body(bmaxa_ref, bmaxb_ref, tids_ref, jsel_ref):
    vals = jnp.concatenate([bmaxa_ref[...], bmaxb_ref[...]], axis=1)
    cio = jax.lax.broadcasted_iota(jnp.int32, (B, NC), 1)
    blkid = jnp.concatenate([2 * cio, 2 * cio + 1], axis=1)
    rowbase = jax.lax.broadcasted_iota(jnp.int32, (B, 1), 0) * NC
    for k in range(K):
        m = jnp.max(vals, axis=1, keepdims=True)
        cand = jnp.where(vals == m, blkid, INT_MAX)
        j = jnp.min(cand, axis=1, keepdims=True)
        jsel_ref[:, k:k + 1] = j
        tids_ref[:, k:k + 1] = rowbase + (j >> 1)
        vals = jnp.where(blkid == j, NEG_INF, vals)


def _k2_select_blocks(bmaxa, bmaxb):
    return pl.pallas_call(
        _k2_body,
        in_specs=[
            pl.BlockSpec((B, NC), lambda: (0, 0)),
            pl.BlockSpec((B, NC), lambda: (0, 0)),
        ],
        out_specs=[
            pl.BlockSpec((B, K), lambda: (0, 0)),
            pl.BlockSpec((B, K), lambda: (0, 0)),
        ],
        out_shape=[
            jax.ShapeDtypeStruct((B, K), jnp.int32),
            jax.ShapeDtypeStruct((B, K), jnp.int32),
        ],
        interpret=False,
    )(bmaxa, bmaxb)


_GATHER_WINDOW = 128
_NUM_IDX_PAD = 32 * _GATHER_WINDOW


def _sc_gather(table, tids_flat):
    num_idx = _NUM_IDX_PAD

    @pl.kernel(
        out_type=jax.ShapeDtypeStruct((num_idx, 2 * L), jnp.float32),
        mesh=plsc.VectorSubcoreMesh(core_axis_name="c", subcore_axis_name="s"),
    )
    def gather_kernel(x_hbm, i_hbm, o_hbm):
        def body(i_vmem, o_vmem):
            pltpu.sync_copy(x_hbm.at[i_vmem.at[0]], o_vmem)

        pltpu.emit_pipeline(
            body,
            grid=(num_idx // _GATHER_WINDOW,),
            in_specs=[pl.BlockSpec((1, _GATHER_WINDOW), lambda i: (0, i))],
            out_specs=[pl.BlockSpec((_GATHER_WINDOW, 2 * L), lambda i: (i, 0))],
            core_axis_name=("c", "s"),
            dimension_semantics=(pltpu.PARALLEL,),
        )(i_hbm, o_hbm)

    return gather_kernel(table, tids_flat)


def _k3_body(cand_ref, jsel_ref, g_ref, probs_ref, s_ref):
    lane128 = jax.lax.broadcasted_iota(jnp.int32, (B, 2 * L), 1)
    vals_cols = []
    gidx_cols = []
    for k in range(K):
        j = jsel_ref[:, k:k + 1]
        h = j & 1
        valid = (lane128 >= L) == (h == 1)
        c = cand_ref[:, k * 2 * L:(k + 1) * 2 * L]
        vals_cols.append(jnp.where(valid, c, NEG_INF))
        gidx_cols.append(jnp.where(valid, (j >> 1) * (2 * L) + lane128, INT_MAX))
    vals = jnp.concatenate(vals_cols, axis=1)
    gidx = jnp.concatenate(gidx_cols, axis=1)

    topv_cols = []
    topi_cols = []
    for k in range(K):
        m = jnp.max(vals, axis=1, keepdims=True)
        sel = jnp.where(vals == m, gidx, INT_MAX)
        gmin = jnp.min(sel, axis=1, keepdims=True)
        topv_cols.append(m)
        topi_cols.append(gmin)
        vals = jnp.where(gidx == gmin, NEG_INF, vals)
    topv = jnp.concatenate(topv_cols, axis=1)
    topi = jnp.concatenate(topi_cols, axis=1)

    e = jnp.exp(topv - topv[:, 0:1])
    probs_ref[...] = e / jnp.sum(e, axis=1, keepdims=True)

    kio = jax.lax.broadcasted_iota(jnp.int32, (B, K), 1)
    for r in range(R):
        z = topv + g_ref[:, r * K:(r + 1) * K]
        mz = jnp.max(z, axis=1, keepdims=True)
        ksel = jnp.where(z == mz, kio, INT_MAX)
        kmin = jnp.min(ksel, axis=1, keepdims=True)
        smp = jnp.sum(jnp.where(kio == kmin, topi, 0), axis=1, keepdims=True)
        s_ref[:, r:r + 1] = smp


def _k3_finalize(cand2, jsel, g2):
    return pl.pallas_call(
        _k3_body,
        in_specs=[
            pl.BlockSpec((B, K * 2 * L), lambda: (0, 0)),
            pl.BlockSpec((B, K), lambda: (0, 0)),
            pl.BlockSpec((B, R * K), lambda: (0, 0)),
        ],
        out_specs=[
            pl.BlockSpec((B, K), lambda: (0, 0)),
            pl.BlockSpec((B, R), lambda: (0, 0)),
        ],
        out_shape=[
            jax.ShapeDtypeStruct((B, K), jnp.float32),
            jax.ShapeDtypeStruct((B, R), jnp.int32),
        ],
        interpret=False,
    )(cand2, jsel, g2)


def kernel(logits):
    assert logits.shape == (B, V)
    bmaxa, bmaxb, table3 = _k1_block_max(logits)
    tids, jsel = _k2_select_blocks(bmaxa, bmaxb)

    table = table3.reshape(B * NC, 2 * L)
    tids_pad = jnp.pad(tids.reshape(1, B * K), ((0, 0), (0, _NUM_IDX_PAD - B * K)))
    cand = _sc_gather(table, tids_pad)[: B * K]

    g = jax.random.gumbel(jax.random.key(42), (R, B, K), dtype=jnp.float32)
    g2 = jnp.transpose(g, (1, 0, 2)).reshape(B, R * K)

    probs, s = _k3_finalize(cand.reshape(B, K * 2 * L), jsel, g2)
    return probs, s.T
